# Initial kernel scaffold; baseline (speedup 1.0000x reference)
import jax
import jax.numpy as jnp
from jax.experimental import pallas as pl


def kernel(x, edge_index, num_nodes, W):
    raise NotImplementedError("write your pallas kernel here")


# speedup vs baseline: 8.2457x; 8.2457x over previous
import functools

import jax
import jax.numpy as jnp
from jax import lax
from jax.experimental import pallas as pl
from jax.experimental.pallas import tpu as pltpu
from jax.experimental.pallas import tpu_sc as plsc

NC, NS, L = 2, 16, 16
NW = NC * NS
N = 10000
NPAD = 10240
SLICE = NPAD // NS
E = 320000
EPT = E // NW
D = 128
CH = 80
NCH = EPT // CH

_mesh = plsc.VectorSubcoreMesh(core_axis_name="c", subcore_axis_name="s")


@functools.partial(
    pl.kernel,
    out_type=jax.ShapeDtypeStruct((NC, NPAD), jnp.float32),
    mesh=_mesh,
    scratch_types=[
        pltpu.VMEM((CH,), jnp.int32),
        pltpu.VMEM((CH,), jnp.float32),
        pltpu.VMEM_SHARED((NPAD,), jnp.float32),
    ],
)
def _degree_kernel(dst_hbm, deg_out, dst_v, ones_v, deg_sp):
    c = lax.axis_index("c")
    s = lax.axis_index("s")
    wid = c * NS + s

    def fill(i, val):
        ones_v[pl.ds(i * L, L)] = jnp.full((L,), val, jnp.float32)
        return val
    lax.fori_loop(0, CH // L, fill, 0.0)
    for k in range(SLICE // CH):
        pltpu.sync_copy(ones_v, deg_sp.at[pl.ds(s * SLICE + k * CH, CH)])
    plsc.subcore_barrier()
    lax.fori_loop(0, CH // L, fill, 1.0)

    def step(i, _):
        off = wid * EPT + i * CH
        pltpu.sync_copy(dst_hbm.at[pl.ds(off, CH)], dst_v)
        pltpu.sync_copy(ones_v, deg_sp.at[dst_v], add=True)
        return 0
    lax.fori_loop(0, NCH, step, 0)

    plsc.subcore_barrier()
    pltpu.sync_copy(deg_sp.at[pl.ds(s * SLICE, SLICE)],
                    deg_out.at[c, pl.ds(s * SLICE, SLICE)])


@functools.partial(
    pl.kernel,
    out_type=jax.ShapeDtypeStruct((NC, NPAD, D), jnp.float32),
    mesh=_mesh,
    scratch_types=[
        pltpu.VMEM((CH,), jnp.int32),
        pltpu.VMEM((CH,), jnp.int32),
        pltpu.VMEM((CH, D), jnp.float32),
        pltpu.VMEM_SHARED((NPAD, D), jnp.float32),
        pltpu.SemaphoreType.DMA,
    ],
)
def _scatter_kernel(src_hbm, dst_hbm, xs_hbm, s_out,
                    src_v, dst_v, rows_v, agg_sp, sem):
    c = lax.axis_index("c")
    s = lax.axis_index("s")
    wid = c * NS + s

    def zr(r, _):
        for j in range(D // L):
            rows_v[r, pl.ds(j * L, L)] = jnp.zeros((L,), jnp.float32)
        return 0
    lax.fori_loop(0, CH, zr, 0)
    for k in range(SLICE // CH):
        pltpu.sync_copy(rows_v, agg_sp.at[pl.ds(s * SLICE + k * CH, CH)])
    plsc.subcore_barrier()

    base = wid * EPT

    def step(i, _):
        off = base + i * CH
        pltpu.sync_copy(src_hbm.at[pl.ds(off, CH)], src_v)
        pltpu.sync_copy(dst_hbm.at[pl.ds(off, CH)], dst_v)
        pltpu.async_copy(xs_hbm.at[src_v], rows_v, sem).wait()
        pltpu.sync_copy(rows_v, agg_sp.at[dst_v], add=True)
        return 0
    lax.fori_loop(0, NCH, step, 0)

    plsc.subcore_barrier()
    pltpu.sync_copy(agg_sp.at[pl.ds(s * SLICE, SLICE)],
                    s_out.at[c, pl.ds(s * SLICE, SLICE)])


def _prescale_body(deg_ref, x_ref, adj_ref, xs_ref, dis_ref):
    deg = deg_ref[0] + deg_ref[1] + 1.0 + adj_ref[0, 0]
    dis = lax.rsqrt(deg)
    dis_ref[...] = dis
    xs_ref[...] = x_ref[...] * dis


def _prescale(deg2, x, adj):
    return pl.pallas_call(
        _prescale_body,
        out_shape=[
            jax.ShapeDtypeStruct((N, D), jnp.float32),
            jax.ShapeDtypeStruct((N, 1), jnp.float32),
        ],
    )(deg2, x, adj)


def _combine_body(s_ref, xs_ref, dis_ref, wt_ref, out_ref):
    agg = s_ref[0] + s_ref[1] + xs_ref[...]
    a = agg * dis_ref[...]
    out_ref[...] = jnp.dot(a, wt_ref[...], preferred_element_type=jnp.float32)


def _combine(s2, xs, dis, wt):
    rb = 400
    grid = N // rb
    return pl.pallas_call(
        _combine_body,
        grid=(grid,),
        in_specs=[
            pl.BlockSpec((NC, rb, D), lambda i: (0, i, 0)),
            pl.BlockSpec((rb, D), lambda i: (i, 0)),
            pl.BlockSpec((rb, 1), lambda i: (i, 0)),
            pl.BlockSpec((D, D), lambda i: (0, 0)),
        ],
        out_specs=pl.BlockSpec((rb, D), lambda i: (i, 0)),
        out_shape=jax.ShapeDtypeStruct((N, D), jnp.float32),
    )(s2, xs, dis, wt)


def kernel(x, edge_index, num_nodes, W):
    src = edge_index[0].astype(jnp.int32)
    dst = edge_index[1].astype(jnp.int32)
    adj = (jnp.asarray(num_nodes, jnp.float32) - x.shape[0]).reshape(1, 1)

    deg_p = _degree_kernel(dst)
    deg2 = deg_p[:, :N, None]
    xs, dis = _prescale(deg2, x, adj)
    s_p = _scatter_kernel(src, dst, xs)
    return _combine(s_p[:, :N, :], xs, dis, W.T)


# speedup vs baseline: 17.5775x; 2.1317x over previous
import functools

import jax
import jax.numpy as jnp
from jax import lax
from jax.experimental import pallas as pl
from jax.experimental.pallas import tpu as pltpu
from jax.experimental.pallas import tpu_sc as plsc

NC, NS, L = 2, 16, 16
NW = NC * NS
N = 10000
NPAD = 10240
SLICE = NPAD // NS
E = 320000
EPT = E // NW
D = 128
CH = 80
NCH = EPT // CH

_mesh = plsc.VectorSubcoreMesh(core_axis_name="c", subcore_axis_name="s")


NB = 25
NBLK = NCH // NB


@functools.partial(
    pl.kernel,
    out_type=jax.ShapeDtypeStruct((NC, NPAD), jnp.float32),
    mesh=_mesh,
    scratch_types=[
        pltpu.VMEM((NB, CH), jnp.int32),
        pltpu.VMEM((NB, CH), jnp.int32),
        pltpu.VMEM((CH,), jnp.float32),
        pltpu.VMEM_SHARED((NPAD,), jnp.float32),
        pltpu.SemaphoreType.DMA,
        pltpu.SemaphoreType.DMA,
    ],
)
def _degree_kernel(dst_hbm, deg_out, dst_b0, dst_b1, ones_v, deg_sp,
                   sem0, sem1):
    c = lax.axis_index("c")
    s = lax.axis_index("s")
    wid = c * NS + s

    def fill(i, val):
        ones_v[pl.ds(i * L, L)] = jnp.full((L,), val, jnp.float32)
        return val
    lax.fori_loop(0, CH // L, fill, 0.0)
    for k in range(SLICE // CH):
        pltpu.sync_copy(ones_v, deg_sp.at[pl.ds(s * SLICE + k * CH, CH)])
    plsc.subcore_barrier()
    lax.fori_loop(0, CH // L, fill, 1.0)

    bufs = (dst_b0, dst_b1)
    sems = (sem0, sem1)

    def fire(buf, sem):
        def one(i, _):
            pltpu.async_copy(ones_v, deg_sp.at[buf.at[i]], sem, add=True)
            return 0
        lax.fori_loop(0, NB, one, 0)

    def drain(buf, sem):
        def one(i, _):
            pltpu.make_async_copy(ones_v, deg_sp.at[buf.at[0]], sem).wait()
            return 0
        lax.fori_loop(0, NB, one, 0)

    for blk in range(NBLK):
        p = blk % 2
        if blk >= 2:
            drain(bufs[p], sems[p])
        pltpu.sync_copy(dst_hbm.at[wid, blk], bufs[p])
        fire(bufs[p], sems[p])
    for blk in range(NBLK - 2, NBLK):
        p = blk % 2
        drain(bufs[p], sems[p])

    plsc.subcore_barrier()
    pltpu.sync_copy(deg_sp.at[pl.ds(s * SLICE, SLICE)],
                    deg_out.at[c, pl.ds(s * SLICE, SLICE)])


@functools.partial(
    pl.kernel,
    out_type=jax.ShapeDtypeStruct((NC, NPAD, D), jnp.float32),
    mesh=_mesh,
    scratch_types=[
        pltpu.VMEM((NB, CH), jnp.int32),
        pltpu.VMEM((NB, CH), jnp.int32),
        pltpu.VMEM((CH, D), jnp.float32),
        pltpu.VMEM((CH, D), jnp.float32),
        pltpu.VMEM_SHARED((NPAD, D), jnp.float32),
        pltpu.SemaphoreType.DMA,
        pltpu.SemaphoreType.DMA,
    ],
)
def _scatter_kernel(src_hbm, dst_hbm, xs_hbm, s_out,
                    src_blk, dst_blk, rows0, rows1, agg_sp, sem0, sem1):
    c = lax.axis_index("c")
    s = lax.axis_index("s")
    wid = c * NS + s

    def zr(r, _):
        for j in range(D // L):
            rows0[r, pl.ds(j * L, L)] = jnp.zeros((L,), jnp.float32)
        return 0
    lax.fori_loop(0, CH, zr, 0)
    for k in range(SLICE // CH):
        pltpu.sync_copy(rows0, agg_sp.at[pl.ds(s * SLICE + k * CH, CH)])
    plsc.subcore_barrier()

    def gather(i, buf, sem):
        pltpu.async_copy(xs_hbm.at[src_blk.at[i]], buf, sem)

    def gwait(buf, sem):
        pltpu.make_async_copy(xs_hbm.at[src_blk.at[0]], buf, sem).wait()

    def scat(i, buf):
        pltpu.sync_copy(buf, agg_sp.at[dst_blk.at[i]], add=True)

    for blk in range(NBLK):
        pltpu.sync_copy(src_hbm.at[wid, blk], src_blk)
        pltpu.sync_copy(dst_hbm.at[wid, blk], dst_blk)
        gather(0, rows0, sem0)

        def pair(j, _):
            a = 2 * j
            gather(a + 1, rows1, sem1)
            gwait(rows0, sem0)
            scat(a, rows0)
            gather(a + 2, rows0, sem0)
            gwait(rows1, sem1)
            scat(a + 1, rows1)
            return 0
        lax.fori_loop(0, (NB - 1) // 2, pair, 0)

        gwait(rows0, sem0)
        scat(NB - 1, rows0)

    plsc.subcore_barrier()
    pltpu.sync_copy(agg_sp.at[pl.ds(s * SLICE, SLICE)],
                    s_out.at[c, pl.ds(s * SLICE, SLICE)])


def _prescale_body(deg_ref, x_ref, adj_ref, xs_ref, dis_ref):
    deg = deg_ref[0] + deg_ref[1] + 1.0 + adj_ref[0, 0]
    dis = lax.rsqrt(deg)
    dis_ref[...] = dis
    xs_ref[...] = x_ref[...] * dis


def _prescale(deg2, x, adj):
    return pl.pallas_call(
        _prescale_body,
        out_shape=[
            jax.ShapeDtypeStruct((N, D), jnp.float32),
            jax.ShapeDtypeStruct((N, 1), jnp.float32),
        ],
    )(deg2, x, adj)


def _combine_body(s_ref, xs_ref, dis_ref, wt_ref, out_ref):
    agg = s_ref[0] + s_ref[1] + xs_ref[...]
    a = agg * dis_ref[...]
    out_ref[...] = jnp.dot(a, wt_ref[...], preferred_element_type=jnp.float32)


def _combine(s2, xs, dis, wt):
    rb = 400
    grid = N // rb
    return pl.pallas_call(
        _combine_body,
        grid=(grid,),
        in_specs=[
            pl.BlockSpec((NC, rb, D), lambda i: (0, i, 0)),
            pl.BlockSpec((rb, D), lambda i: (i, 0)),
            pl.BlockSpec((rb, 1), lambda i: (i, 0)),
            pl.BlockSpec((D, D), lambda i: (0, 0)),
        ],
        out_specs=pl.BlockSpec((rb, D), lambda i: (i, 0)),
        out_shape=jax.ShapeDtypeStruct((N, D), jnp.float32),
    )(s2, xs, dis, wt)


def kernel(x, edge_index, num_nodes, W):
    src4 = edge_index[0].astype(jnp.int32).reshape(NW, NBLK, NB, CH)
    dst4 = edge_index[1].astype(jnp.int32).reshape(NW, NBLK, NB, CH)
    adj = (jnp.asarray(num_nodes, jnp.float32) - x.shape[0]).reshape(1, 1)

    deg_p = _degree_kernel(dst4)
    deg2 = deg_p[:, :N, None]
    xs, dis = _prescale(deg2, x, adj)
    s_p = _scatter_kernel(src4, dst4, xs)
    return _combine(s_p[:, :N, :], xs, dis, W.T)
